# baseline (device time: 186214 ns/iter reference)
import jax
import jax.numpy as jnp
from jax import lax
from jax.experimental import pallas as pl
from jax.experimental.pallas import tpu as pltpu


def kernel(O, Wo):
    B, S, H, D = O.shape
    HD = H * D
    N = Wo.shape[1]
    s_half = S // 2
    n_half = N // 2
    s_sub = s_half // 2
    n_pair = B // 2

    O = O.reshape(B, S, HD)

    def body(o_hbm, w_hbm, out_hbm,
             w_stage, w_buf, o_tile, send_buf, rx_buf,
             sx_sems, rx_sems, sy_sems, ry_sems,
             w_sems, load_sem, store_sems):
        my_x = lax.axis_index("x")
        my_y = lax.axis_index("y")
        x_peer = (1 - my_x, my_y)
        y_peer = (my_x, 1 - my_y)

        peer_s0 = (1 - my_x) * s_half
        my_s0 = my_x * s_half
        col0 = my_y * n_half

        def load_pair(m, s0):
            cp = pltpu.make_async_copy(
                o_hbm.at[pl.ds(2 * m, 2), pl.ds(s0, s_half), :],
                o_tile, load_sem,
            )
            cp.start()
            return cp

        n_wc = 4
        wc_rows = HD // n_wc
        wcps = []
        for c in range(2):
            wcp = pltpu.make_async_copy(
                w_hbm.at[pl.ds(c * wc_rows, wc_rows), pl.ds(col0, n_half)],
                w_stage.at[c], w_sems.at[c],
            )
            wcp.start()
            wcps.append(wcp)
        o_pending = load_pair(0, peer_s0)

        barrier = pltpu.get_barrier_semaphore()
        for nbr in (x_peer, y_peer):
            pl.semaphore_signal(
                barrier, inc=1, device_id=nbr,
                device_id_type=pl.DeviceIdType.MESH,
            )
        pl.semaphore_wait(barrier, 2)

        for c in range(n_wc):
            if c + 2 < n_wc:
                nxt = pltpu.make_async_copy(
                    w_hbm.at[pl.ds((c + 2) * wc_rows, wc_rows),
                             pl.ds(col0, n_half)],
                    w_stage.at[c % 2], w_sems.at[c % 2],
                )
            wcps[c % 2].wait()
            w_buf[c * wc_rows:(c + 1) * wc_rows, :] = (
                w_stage[c % 2].astype(jnp.bfloat16))
            if c + 2 < n_wc:
                nxt.start()
                wcps[c % 2] = nxt

        x_rdmas = []
        for m in range(n_pair):
            o_pending.wait()
            o_pair = o_tile[...].astype(jnp.bfloat16).reshape(2 * s_half, HD)
            if m + 1 < n_pair:
                o_pending = load_pair(m + 1, peer_s0)
            p = jnp.dot(o_pair, w_buf[...],
                        preferred_element_type=jnp.float32)
            for j in range(2):
                b = 2 * m + j
                send_buf[b, :, :] = (
                    p[j * s_half:(j + 1) * s_half, :].astype(jnp.bfloat16))
                rdma = pltpu.make_async_remote_copy(
                    src_ref=send_buf.at[b],
                    dst_ref=rx_buf.at[b],
                    send_sem=sx_sems.at[b],
                    recv_sem=rx_sems.at[b],
                    device_id=x_peer,
                    device_id_type=pl.DeviceIdType.MESH,
                )
                rdma.start()
                x_rdmas.append(rdma)
            if m + 1 == n_pair:
                o_pending = load_pair(0, my_s0)

        y_rdmas = []
        stores = []
        for m in range(n_pair):
            o_pending.wait()
            o_pair = o_tile[...].astype(jnp.bfloat16).reshape(2 * s_half, HD)
            if m + 1 < n_pair:
                o_pending = load_pair(m + 1, my_s0)
            p = jnp.dot(o_pair, w_buf[...],
                        preferred_element_type=jnp.float32)
            for j in range(2):
                b = 2 * m + j
                x_rdmas[b].wait()
                for k in range(2):
                    rows = pl.ds(k * s_sub, s_sub)
                    r0 = j * s_half + k * s_sub
                    send_buf[b, rows, :] = (
                        p[r0:r0 + s_sub, :]
                        + rx_buf[b, rows, :].astype(jnp.float32)
                    ).astype(jnp.bfloat16)
                    rdma = pltpu.make_async_remote_copy(
                        src_ref=send_buf.at[b, rows],
                        dst_ref=out_hbm.at[b, rows, pl.ds(col0, n_half)],
                        send_sem=sy_sems.at[2 * b + k],
                        recv_sem=ry_sems.at[2 * b + k],
                        device_id=y_peer,
                        device_id_type=pl.DeviceIdType.MESH,
                    )
                    rdma.start()
                    y_rdmas.append(rdma)
                st = pltpu.make_async_copy(
                    send_buf.at[b], out_hbm.at[b, :, pl.ds(col0, n_half)],
                    store_sems.at[b],
                )
                st.start()
                stores.append(st)

        for rdma in y_rdmas:
            rdma.wait()
        for st in stores:
            st.wait()

    out = pl.pallas_call(
        body,
        out_shape=jax.ShapeDtypeStruct((B, s_half, N), jnp.bfloat16),
        in_specs=[
            pl.BlockSpec(memory_space=pl.ANY),
            pl.BlockSpec(memory_space=pl.ANY),
        ],
        out_specs=pl.BlockSpec(memory_space=pl.ANY),
        scratch_shapes=[
            pltpu.VMEM((2, HD // 4, n_half), jnp.float32),
            pltpu.VMEM((HD, n_half), jnp.bfloat16),
            pltpu.VMEM((2, s_half, HD), jnp.float32),
            pltpu.VMEM((B, s_half, n_half), jnp.bfloat16),
            pltpu.VMEM((B, s_half, n_half), jnp.bfloat16),
            pltpu.SemaphoreType.DMA((B,)),
            pltpu.SemaphoreType.DMA((B,)),
            pltpu.SemaphoreType.DMA((2 * B,)),
            pltpu.SemaphoreType.DMA((2 * B,)),
            pltpu.SemaphoreType.DMA((2,)),
            pltpu.SemaphoreType.DMA,
            pltpu.SemaphoreType.DMA((B,)),
        ],
        compiler_params=pltpu.CompilerParams(
            collective_id=0,
            vmem_limit_bytes=100 * 1024 * 1024,
        ),
    )(O, Wo)
    return out


# device time: 179318 ns/iter; 1.0385x vs baseline; 1.0385x over previous
import jax
import jax.numpy as jnp
from jax import lax
from jax.experimental import pallas as pl
from jax.experimental.pallas import tpu as pltpu


def kernel(O, Wo):
    B, S, H, D = O.shape
    HD = H * D
    N = Wo.shape[1]
    s_half = S // 2
    n_half = N // 2
    hd_half = HD // 2
    s_sub = s_half // 2

    O = O.reshape(B, S, HD)

    def body(o_hbm, w_hbm, out_hbm,
             w_stage, w_buf, o_slots, send_buf, rx_buf,
             sx_sems, rx_sems, sy_sems, ry_sems,
             w_sems, load_sems, store_sems):
        my_x = lax.axis_index("x")
        my_y = lax.axis_index("y")
        x_peer = (1 - my_x, my_y)
        y_peer = (my_x, 1 - my_y)

        peer_s0 = (1 - my_x) * s_half
        my_s0 = my_x * s_half
        col0 = my_y * n_half

        loads = [(b, peer_s0) for b in range(B)] + [(b, my_s0) for b in range(B)]

        def start_load(i):
            b, s0 = loads[i]
            cp = pltpu.make_async_copy(
                o_hbm.at[b, pl.ds(s0, s_half), :],
                o_slots.at[i % 2], load_sems.at[i % 2],
            )
            cp.start()
            return cp

        wcps = []
        for c in range(2):
            wcp = pltpu.make_async_copy(
                w_hbm.at[pl.ds(c * hd_half, hd_half), pl.ds(col0, n_half)],
                w_stage.at[c], w_sems.at[c],
            )
            wcp.start()
            wcps.append(wcp)
        pending = {0: start_load(0)}

        barrier = pltpu.get_barrier_semaphore()
        for nbr in (x_peer, y_peer):
            pl.semaphore_signal(
                barrier, inc=1, device_id=nbr,
                device_id_type=pl.DeviceIdType.MESH,
            )
        pl.semaphore_wait(barrier, 2)

        for c in range(2):
            wcps[c].wait()
            w_buf[c * hd_half:(c + 1) * hd_half, :] = (
                w_stage[c].astype(jnp.bfloat16))

        x_rdmas = []
        for b in range(B):
            pending[b + 1] = start_load(b + 1)
            pending.pop(b).wait()
            p = jnp.dot(o_slots[b % 2].astype(jnp.bfloat16), w_buf[...],
                        preferred_element_type=jnp.float32)
            send_buf[b, :, :] = p.astype(jnp.bfloat16)
            rdma = pltpu.make_async_remote_copy(
                src_ref=send_buf.at[b],
                dst_ref=rx_buf.at[b],
                send_sem=sx_sems.at[b],
                recv_sem=rx_sems.at[b],
                device_id=x_peer,
                device_id_type=pl.DeviceIdType.MESH,
            )
            rdma.start()
            x_rdmas.append(rdma)

        y_rdmas = []
        stores = []
        for b in range(B):
            i = B + b
            if i + 1 < 2 * B:
                pending[i + 1] = start_load(i + 1)
            pending.pop(i).wait()
            p = jnp.dot(o_slots[i % 2].astype(jnp.bfloat16), w_buf[...],
                        preferred_element_type=jnp.float32)
            x_rdmas[b].wait()
            for k in range(2):
                rows = pl.ds(k * s_sub, s_sub)
                send_buf[b, rows, :] = (
                    p[k * s_sub:(k + 1) * s_sub, :]
                    + rx_buf[b, rows, :].astype(jnp.float32)
                ).astype(jnp.bfloat16)
                rdma = pltpu.make_async_remote_copy(
                    src_ref=send_buf.at[b, rows],
                    dst_ref=out_hbm.at[b, rows, pl.ds(col0, n_half)],
                    send_sem=sy_sems.at[2 * b + k],
                    recv_sem=ry_sems.at[2 * b + k],
                    device_id=y_peer,
                    device_id_type=pl.DeviceIdType.MESH,
                )
                rdma.start()
                y_rdmas.append(rdma)
            st = pltpu.make_async_copy(
                send_buf.at[b], out_hbm.at[b, :, pl.ds(col0, n_half)],
                store_sems.at[b],
            )
            st.start()
            stores.append(st)

        for rdma in y_rdmas:
            rdma.wait()
        for st in stores:
            st.wait()

    out = pl.pallas_call(
        body,
        out_shape=jax.ShapeDtypeStruct((B, s_half, N), jnp.bfloat16),
        in_specs=[
            pl.BlockSpec(memory_space=pl.ANY),
            pl.BlockSpec(memory_space=pl.ANY),
        ],
        out_specs=pl.BlockSpec(memory_space=pl.ANY),
        scratch_shapes=[
            pltpu.VMEM((2, hd_half, n_half), jnp.float32),
            pltpu.VMEM((HD, n_half), jnp.bfloat16),
            pltpu.VMEM((2, s_half, HD), jnp.float32),
            pltpu.VMEM((B, s_half, n_half), jnp.bfloat16),
            pltpu.VMEM((B, s_half, n_half), jnp.bfloat16),
            pltpu.SemaphoreType.DMA((B,)),
            pltpu.SemaphoreType.DMA((B,)),
            pltpu.SemaphoreType.DMA((2 * B,)),
            pltpu.SemaphoreType.DMA((2 * B,)),
            pltpu.SemaphoreType.DMA((2,)),
            pltpu.SemaphoreType.DMA((2,)),
            pltpu.SemaphoreType.DMA((B,)),
        ],
        compiler_params=pltpu.CompilerParams(
            collective_id=0,
            vmem_limit_bytes=100 * 1024 * 1024,
        ),
    )(O, Wo)
    return out
